# baseline (speedup 1.0000x reference)
import functools

import jax
import jax.numpy as jnp
from jax import lax
from jax.experimental import pallas as pl
from jax.experimental.pallas import tpu as pltpu
from jax.experimental.pallas import tpu_sc as plsc

D = 64
EPS = 1e-5
NW = 32
P = 256


def _tables_body(day_ref, mon_ref, dg_ref, db_ref, mg_ref, mb_ref, w_ref,
                 pb_ref, apair_ref, bpair_ref):
    def ln(x, g, b):
        mu = jnp.mean(x, axis=-1, keepdims=True)
        var = jnp.mean((x - mu) ** 2, axis=-1, keepdims=True)
        return (x - mu) / jnp.sqrt(var + EPS) * g + b

    dn = ln(day_ref[...], dg_ref[...], db_ref[...])
    mn = ln(mon_ref[...], mg_ref[...], mb_ref[...])
    a = jnp.dot(dn, w_ref[0:D, :], preferred_element_type=jnp.float32)
    b2 = jnp.dot(mn, w_ref[D:2 * D, :],
                 preferred_element_type=jnp.float32) + pb_ref[...]

    def pair(tbl, n, out_ref):
        rows = lax.broadcasted_iota(jnp.int32, (n * n, n), 0)
        cols = lax.broadcasted_iota(jnp.int32, (n * n, n), 1)
        e1 = (rows // n == cols).astype(jnp.float32)
        e2 = (rows % n == cols).astype(jnp.float32)
        out_ref[...] = jnp.concatenate(
            [jnp.dot(e1, tbl, preferred_element_type=jnp.float32),
             jnp.dot(e2, tbl, preferred_element_type=jnp.float32)], axis=-1)

    pair(a, 32, apair_ref)
    pair(b2, 16, bpair_ref)


def _build_tables(day_w, month_w, day_g, day_b, month_g, month_b, proj_W,
                  proj_b):
    day_p = jnp.pad(day_w.astype(jnp.float32), ((0, 1), (0, 0)))
    mon_p = jnp.pad(month_w.astype(jnp.float32), ((0, 4), (0, 0)))
    return pl.pallas_call(
        _tables_body,
        out_shape=[jax.ShapeDtypeStruct((1024, 2 * D), jnp.float32),
                   jax.ShapeDtypeStruct((256, 2 * D), jnp.float32)],
    )(day_p, mon_p,
      day_g.reshape(1, D), day_b.reshape(1, D),
      month_g.reshape(1, D), month_b.reshape(1, D),
      proj_W, proj_b.reshape(1, D))


def _idx_block(xwv, ia, ib, xoff, j):
    w = xwv[pl.ds(xoff + j * 16, 16)]
    d1 = w & 0xFF
    m1 = lax.shift_right_logical(w, 8) & 0xFF
    d2 = lax.shift_right_logical(w, 16) & 0xFF
    m2 = lax.shift_right_logical(w, 24)
    di1 = jnp.maximum(d1 - 1, 0)
    di2 = jnp.maximum(d2 - 1, 0)
    mi1 = jnp.where((m1 >= 1) & (m1 <= 12), m1 - 1, 0)
    mi2 = jnp.where((m2 >= 1) & (m2 <= 12), m2 - 1, 0)
    ia[j // 8, pl.ds((j % 8) * 16, 16)] = di1 * 32 + di2
    ib[j // 8, pl.ds((j % 8) * 16, 16)] = mi1 * 16 + mi2


def _sc_gather(xw, apair, bpair, npairs):
    per_w = npairs // NW
    nbody = per_w // (2 * P)
    nidx = P // 128
    mesh = plsc.VectorSubcoreMesh(core_axis_name="c", subcore_axis_name="s")

    @functools.partial(
        pl.kernel,
        out_type=jax.ShapeDtypeStruct((npairs, 2 * D), jnp.float32),
        mesh=mesh,
        compiler_params=pltpu.CompilerParams(needs_layout_passes=False),
        scratch_types=[
            pltpu.VMEM_SHARED((1024, 2 * D), jnp.float32),
            pltpu.VMEM_SHARED((256, 2 * D), jnp.float32),
            pltpu.VMEM((4 * P,), jnp.int32),
            pltpu.VMEM((nidx, 128), jnp.int32),
            pltpu.VMEM((nidx, 128), jnp.int32),
            pltpu.VMEM((nidx, 128), jnp.int32),
            pltpu.VMEM((nidx, 128), jnp.int32),
            pltpu.VMEM((P, 2 * D), jnp.float32),
            pltpu.VMEM((P, 2 * D), jnp.float32),
            pltpu.SemaphoreType.DMA,
            pltpu.SemaphoreType.DMA,
            pltpu.SemaphoreType.DMA,
        ],
    )
    def k(xw_hbm, ta_hbm, tb_hbm, out_hbm, ta_sp, tb_sp,
          xwv, ia0, ib0, ia1, ib1, rows0, rows1, sem_g, sem_o, sem_x):
        sid = lax.axis_index("s")

        @pl.when(sid == 0)
        def _():
            pltpu.sync_copy(ta_hbm, ta_sp)
            pltpu.sync_copy(tb_hbm, tb_sp)
        plsc.subcore_barrier()

        wid = sid * 2 + lax.axis_index("c")
        basep = wid * per_w

        pltpu.async_copy(xw_hbm.at[pl.ds(basep, 2 * P)], xwv.at[pl.ds(0, 2 * P)], sem_x)

        def body(g, carry):
            pb0 = basep + g * (2 * P)
            pb1 = pb0 + P

            @pl.when(g > 0)
            def _():
                pltpu.make_async_copy(
                    rows0, out_hbm.at[pl.ds(pb0 - 2 * P, P)], sem_o).wait()
                pltpu.make_async_copy(
                    rows1, out_hbm.at[pl.ds(pb1 - 2 * P, P)], sem_o).wait()

            xo = (g % 2) * (2 * P)
            pltpu.make_async_copy(
                xw_hbm.at[pl.ds(pb0, 2 * P)], xwv.at[pl.ds(xo, 2 * P)], sem_x).wait()

            @pl.when(g + 1 < nbody)
            def _():
                pltpu.async_copy(
                    xw_hbm.at[pl.ds(pb0 + 2 * P, 2 * P)],
                    xwv.at[pl.ds(2 * P - xo, 2 * P)], sem_x)

            for j in range(P // 16):
                _idx_block(xwv, ia0, ib0, xo, j)
            for j in range(P // 16):
                _idx_block(xwv, ia1, ib1, xo + P, j)

            cpa0 = [pltpu.async_copy(ta_sp.at[ia0.at[r]],
                                     rows0.at[pl.ds(r * 128, 128)], sem_g)
                    for r in range(nidx)]
            cpa1 = [pltpu.async_copy(ta_sp.at[ia1.at[r]],
                                     rows1.at[pl.ds(r * 128, 128)], sem_g)
                    for r in range(nidx)]
            cpb0 = []
            for r in range(nidx):
                cpa0[r].wait()
                cpb0.append(pltpu.async_copy(
                    tb_sp.at[ib0.at[r]], rows0.at[pl.ds(r * 128, 128)],
                    sem_g, add=True))
            cpb1 = []
            for r in range(nidx):
                cpb0[r].wait()
                cpa1[r].wait()
                cpb1.append(pltpu.async_copy(
                    tb_sp.at[ib1.at[r]], rows1.at[pl.ds(r * 128, 128)],
                    sem_g, add=True))
            pltpu.async_copy(rows0, out_hbm.at[pl.ds(pb0, P)], sem_o)
            for cp in cpb1:
                cp.wait()
            pltpu.async_copy(rows1, out_hbm.at[pl.ds(pb1, P)], sem_o)
            return carry

        lax.fori_loop(0, nbody, body, 0)
        pltpu.make_async_copy(
            rows0, out_hbm.at[pl.ds(basep + per_w - 2 * P, P)], sem_o).wait()
        pltpu.make_async_copy(
            rows1, out_hbm.at[pl.ds(basep + per_w - P, P)], sem_o).wait()

    return k(xw, apair, bpair)


def kernel(x_mark, hour_w, day_w, month_w, hour_g, hour_b, day_g, day_b,
           month_g, month_b, proj_W, proj_b):
    bsz, seq, _ = x_mark.shape
    n = bsz * seq
    npairs = n // 2
    assert npairs % (NW * 2 * P) == 0
    x8 = x_mark.astype(jnp.int32)
    w = x8[:, :, 1] | (x8[:, :, 2] << 8)
    wp = w.reshape(bsz, seq // 2, 2)
    xw = (wp[:, :, 0] | (wp[:, :, 1] << 16)).reshape(npairs)
    apair, bpair = _build_tables(day_w, month_w, day_g, day_b, month_g,
                                 month_b, proj_W, proj_b)
    out = _sc_gather(xw, apair, bpair, npairs)
    return out.reshape(bsz, seq, D)


# speedup vs baseline: 1.0536x; 1.0536x over previous
import functools

import jax
import jax.numpy as jnp
from jax import lax
from jax.experimental import pallas as pl
from jax.experimental.pallas import tpu as pltpu
from jax.experimental.pallas import tpu_sc as plsc

D = 64
EPS = 1e-5
NW = 32
P = 256


def _tables_body(day_ref, mon_ref, dg_ref, db_ref, mg_ref, mb_ref, w_ref,
                 pb_ref, apair_ref, bpair_ref):
    def ln(x, g, b):
        mu = jnp.mean(x, axis=-1, keepdims=True)
        var = jnp.mean((x - mu) ** 2, axis=-1, keepdims=True)
        return (x - mu) / jnp.sqrt(var + EPS) * g + b

    dn = ln(day_ref[...], dg_ref[...], db_ref[...])
    mn = ln(mon_ref[...], mg_ref[...], mb_ref[...])
    a = jnp.dot(dn, w_ref[0:D, :], preferred_element_type=jnp.float32)
    b2 = jnp.dot(mn, w_ref[D:2 * D, :],
                 preferred_element_type=jnp.float32) + pb_ref[...]

    def pair(tbl, n, out_ref):
        rows = lax.broadcasted_iota(jnp.int32, (n * n, n), 0)
        cols = lax.broadcasted_iota(jnp.int32, (n * n, n), 1)
        e1 = (rows // n == cols).astype(jnp.float32)
        e2 = (rows % n == cols).astype(jnp.float32)
        out_ref[...] = jnp.concatenate(
            [jnp.dot(e1, tbl, preferred_element_type=jnp.float32),
             jnp.dot(e2, tbl, preferred_element_type=jnp.float32)], axis=-1)

    pair(a, 32, apair_ref)
    pair(b2, 16, bpair_ref)


def _build_tables(day_w, month_w, day_g, day_b, month_g, month_b, proj_W,
                  proj_b):
    day_p = jnp.pad(day_w.astype(jnp.float32), ((0, 1), (0, 0)))
    mon_p = jnp.pad(month_w.astype(jnp.float32), ((0, 4), (0, 0)))
    return pl.pallas_call(
        _tables_body,
        out_shape=[jax.ShapeDtypeStruct((1024, 2 * D), jnp.float32),
                   jax.ShapeDtypeStruct((256, 2 * D), jnp.float32)],
    )(day_p, mon_p,
      day_g.reshape(1, D), day_b.reshape(1, D),
      month_g.reshape(1, D), month_b.reshape(1, D),
      proj_W, proj_b.reshape(1, D))


def _idx_block(xwv, ia, ib, xoff, j):
    w = xwv[pl.ds(xoff + j * 16, 16)]
    d1 = w & 0xFF
    m1 = lax.shift_right_logical(w, 8) & 0xFF
    d2 = lax.shift_right_logical(w, 16) & 0xFF
    m2 = lax.shift_right_logical(w, 24)
    di1 = jnp.maximum(d1 - 1, 0)
    di2 = jnp.maximum(d2 - 1, 0)
    mi1 = jnp.where((m1 >= 1) & (m1 <= 12), m1 - 1, 0)
    mi2 = jnp.where((m2 >= 1) & (m2 <= 12), m2 - 1, 0)
    ia[j // 8, pl.ds((j % 8) * 16, 16)] = di1 * 32 + di2
    ib[j // 8, pl.ds((j % 8) * 16, 16)] = mi1 * 16 + mi2


def _sc_gather(xw, apair, bpair, npairs):
    per_w = npairs // NW
    nbody = per_w // (2 * P)
    nidx = P // 128
    mesh = plsc.VectorSubcoreMesh(core_axis_name="c", subcore_axis_name="s")

    @functools.partial(
        pl.kernel,
        out_type=jax.ShapeDtypeStruct((npairs, 2 * D), jnp.float32),
        mesh=mesh,
        compiler_params=pltpu.CompilerParams(needs_layout_passes=False),
        scratch_types=[
            pltpu.VMEM_SHARED((1024, 2 * D), jnp.float32),
            pltpu.VMEM_SHARED((256, 2 * D), jnp.float32),
            pltpu.VMEM((4 * P,), jnp.int32),
            pltpu.VMEM((nidx, 128), jnp.int32),
            pltpu.VMEM((nidx, 128), jnp.int32),
            pltpu.VMEM((nidx, 128), jnp.int32),
            pltpu.VMEM((nidx, 128), jnp.int32),
            pltpu.VMEM((P, 2 * D), jnp.float32),
            pltpu.VMEM((P, 2 * D), jnp.float32),
            pltpu.SemaphoreType.DMA,
            pltpu.SemaphoreType.DMA,
            pltpu.SemaphoreType.DMA,
        ],
    )
    def k(xw_hbm, ta_hbm, tb_hbm, out_hbm, ta_sp, tb_sp,
          xwv, ia0, ib0, ia1, ib1, rows0, rows1, sem_g, sem_o, sem_x):
        sid = lax.axis_index("s")

        @pl.when(sid == 0)
        def _():
            pltpu.sync_copy(ta_hbm, ta_sp)
            pltpu.sync_copy(tb_hbm, tb_sp)
        plsc.subcore_barrier()

        wid = sid * 2 + lax.axis_index("c")
        basep = wid * per_w

        pltpu.async_copy(xw_hbm.at[pl.ds(basep, 2 * P)], xwv.at[pl.ds(0, 2 * P)], sem_x)

        def body(g, carry):
            pb0 = basep + g * (2 * P)
            pb1 = pb0 + P

            @pl.when(g > 0)
            def _():
                pltpu.make_async_copy(
                    rows0, out_hbm.at[pl.ds(pb0 - 2 * P, P)], sem_o).wait()
                pltpu.make_async_copy(
                    rows1, out_hbm.at[pl.ds(pb1 - 2 * P, P)], sem_o).wait()

            xo = (g % 2) * (2 * P)
            pltpu.make_async_copy(
                xw_hbm.at[pl.ds(pb0, 2 * P)], xwv.at[pl.ds(xo, 2 * P)], sem_x).wait()

            @pl.when(g + 1 < nbody)
            def _():
                pltpu.async_copy(
                    xw_hbm.at[pl.ds(pb0 + 2 * P, 2 * P)],
                    xwv.at[pl.ds(2 * P - xo, 2 * P)], sem_x)

            for j in range(P // 16):
                _idx_block(xwv, ia0, ib0, xo, j)
            for j in range(P // 16):
                _idx_block(xwv, ia1, ib1, xo + P, j)

            cpa0 = [pltpu.async_copy(ta_sp.at[ia0.at[r]],
                                     rows0.at[pl.ds(r * 128, 128)], sem_g)
                    for r in range(nidx)]
            for cp in cpa0:
                cp.wait()
            cpb0 = [pltpu.async_copy(tb_sp.at[ib0.at[r]],
                                     rows0.at[pl.ds(r * 128, 128)], sem_g,
                                     add=True)
                    for r in range(nidx)]
            cpa1 = [pltpu.async_copy(ta_sp.at[ia1.at[r]],
                                     rows1.at[pl.ds(r * 128, 128)], sem_g)
                    for r in range(nidx)]
            for cp in cpb0:
                cp.wait()
            pltpu.async_copy(rows0, out_hbm.at[pl.ds(pb0, P)], sem_o)
            for cp in cpa1:
                cp.wait()
            cpb1 = [pltpu.async_copy(tb_sp.at[ib1.at[r]],
                                     rows1.at[pl.ds(r * 128, 128)], sem_g,
                                     add=True)
                    for r in range(nidx)]
            for cp in cpb1:
                cp.wait()
            pltpu.async_copy(rows1, out_hbm.at[pl.ds(pb1, P)], sem_o)
            return carry

        lax.fori_loop(0, nbody, body, 0)
        pltpu.make_async_copy(
            rows0, out_hbm.at[pl.ds(basep + per_w - 2 * P, P)], sem_o).wait()
        pltpu.make_async_copy(
            rows1, out_hbm.at[pl.ds(basep + per_w - P, P)], sem_o).wait()

    return k(xw, apair, bpair)


def kernel(x_mark, hour_w, day_w, month_w, hour_g, hour_b, day_g, day_b,
           month_g, month_b, proj_W, proj_b):
    bsz, seq, _ = x_mark.shape
    n = bsz * seq
    npairs = n // 2
    assert npairs % (NW * 2 * P) == 0
    x8 = x_mark.astype(jnp.int32)
    w = x8[:, :, 1] | (x8[:, :, 2] << 8)
    wp = w.reshape(bsz, seq // 2, 2)
    xw = (wp[:, :, 0] | (wp[:, :, 1] << 16)).reshape(npairs)
    apair, bpair = _build_tables(day_w, month_w, day_g, day_b, month_g,
                                 month_b, proj_W, proj_b)
    out = _sc_gather(xw, apair, bpair, npairs)
    return out.reshape(bsz, seq, D)
